# initial kernel scaffold (unmeasured)
import jax
import jax.numpy as jnp
from jax import lax
from jax.experimental import pallas as pl
from jax.experimental.pallas import tpu as pltpu


def kernel(
    x,
):
    def body(*refs):
        pass

    out_shape = jax.ShapeDtypeStruct(..., jnp.float32)
    return pl.pallas_call(body, out_shape=out_shape)(...)


# baseline (device time: 11415 ns/iter reference)
import jax
import jax.numpy as jnp
from jax import lax
from jax.experimental import pallas as pl
from jax.experimental.pallas import tpu as pltpu

N_DEV = 32


def kernel(x):
    m, n = x.shape

    def body(x_ref, out_ref, halo_ref, send_sems, recv_sems):
        my = lax.axis_index("i")
        left = lax.rem(my + N_DEV - 1, N_DEV)
        right = lax.rem(my + 1, N_DEV)

        barrier_sem = pltpu.get_barrier_semaphore()
        for nbr in (left, right):
            pl.semaphore_signal(
                barrier_sem, inc=1,
                device_id=(nbr,), device_id_type=pl.DeviceIdType.MESH,
            )
        pl.semaphore_wait(barrier_sem, 2)

        rdma_r = pltpu.make_async_remote_copy(
            src_ref=x_ref.at[pl.ds(m - 1, 1)],
            dst_ref=halo_ref.at[0],
            send_sem=send_sems.at[0],
            recv_sem=recv_sems.at[0],
            device_id=(right,),
            device_id_type=pl.DeviceIdType.MESH,
        )
        rdma_l = pltpu.make_async_remote_copy(
            src_ref=x_ref.at[pl.ds(0, 1)],
            dst_ref=halo_ref.at[1],
            send_sem=send_sems.at[1],
            recv_sem=recv_sems.at[1],
            device_id=(left,),
            device_id_type=pl.DeviceIdType.MESH,
        )
        rdma_r.start()
        rdma_l.start()
        rdma_r.wait()
        rdma_l.wait()

        xv = x_ref[:, :]
        up = jnp.concatenate([halo_ref[0], xv[:-1]], axis=0)
        down = jnp.concatenate([xv[1:], halo_ref[1]], axis=0)
        out = 0.25 * up + 0.5 * xv + 0.25 * down

        row = lax.broadcasted_iota(jnp.int32, (m, n), 0)
        fix = ((my == 0) & (row == 0)) | ((my == N_DEV - 1) & (row == m - 1))
        out_ref[:, :] = jnp.where(fix, xv, out)

    return pl.pallas_call(
        body,
        out_shape=jax.ShapeDtypeStruct((m, n), x.dtype),
        in_specs=[pl.BlockSpec(memory_space=pltpu.VMEM)],
        out_specs=pl.BlockSpec(memory_space=pltpu.VMEM),
        scratch_shapes=[
            pltpu.VMEM((2, 1, n), x.dtype),
            pltpu.SemaphoreType.DMA((2,)),
            pltpu.SemaphoreType.DMA((2,)),
        ],
        compiler_params=pltpu.CompilerParams(collective_id=0),
    )(x)


# device time: 11371 ns/iter; 1.0039x vs baseline; 1.0039x over previous
import jax
import jax.numpy as jnp
from jax import lax
from jax.experimental import pallas as pl
from jax.experimental.pallas import tpu as pltpu

N_DEV = 32


def kernel(x):
    m, n = x.shape

    def body(x_ref, out_ref, halo_ref, send_sems, recv_sems):
        my = lax.axis_index("i")
        left = lax.rem(my + N_DEV - 1, N_DEV)
        right = lax.rem(my + 1, N_DEV)

        barrier_sem = pltpu.get_barrier_semaphore()
        for nbr in (left, right):
            pl.semaphore_signal(
                barrier_sem, inc=1,
                device_id=(nbr,), device_id_type=pl.DeviceIdType.MESH,
            )
        pl.semaphore_wait(barrier_sem, 2)

        rdma_r = pltpu.make_async_remote_copy(
            src_ref=x_ref.at[pl.ds(m - 1, 1)],
            dst_ref=halo_ref.at[0],
            send_sem=send_sems.at[0],
            recv_sem=recv_sems.at[0],
            device_id=(right,),
            device_id_type=pl.DeviceIdType.MESH,
        )
        rdma_l = pltpu.make_async_remote_copy(
            src_ref=x_ref.at[pl.ds(0, 1)],
            dst_ref=halo_ref.at[1],
            send_sem=send_sems.at[1],
            recv_sem=recv_sems.at[1],
            device_id=(left,),
            device_id_type=pl.DeviceIdType.MESH,
        )
        rdma_r.start()
        rdma_l.start()

        xv = x_ref[:, :]
        up = jnp.concatenate([xv[:1], xv[:-1]], axis=0)
        down = jnp.concatenate([xv[1:], xv[m - 1:]], axis=0)
        out_ref[:, :] = 0.25 * up + 0.5 * xv + 0.25 * down

        rdma_r.wait_recv()
        rdma_l.wait_recv()

        row0 = 0.25 * halo_ref[0] + 0.5 * xv[:1] + 0.25 * xv[1:2]
        out_ref[pl.ds(0, 1), :] = jnp.where(my == 0, xv[:1], row0)
        rowm = 0.25 * xv[m - 2:m - 1] + 0.5 * xv[m - 1:] + 0.25 * halo_ref[1]
        out_ref[pl.ds(m - 1, 1), :] = jnp.where(my == N_DEV - 1, xv[m - 1:], rowm)

        rdma_r.wait_send()
        rdma_l.wait_send()

    return pl.pallas_call(
        body,
        out_shape=jax.ShapeDtypeStruct((m, n), x.dtype),
        in_specs=[pl.BlockSpec(memory_space=pltpu.VMEM)],
        out_specs=pl.BlockSpec(memory_space=pltpu.VMEM),
        scratch_shapes=[
            pltpu.VMEM((2, 1, n), x.dtype),
            pltpu.SemaphoreType.DMA((2,)),
            pltpu.SemaphoreType.DMA((2,)),
        ],
        compiler_params=pltpu.CompilerParams(collective_id=0),
    )(x)


# device time: 6162 ns/iter; 1.8525x vs baseline; 1.8453x over previous
import jax
import jax.numpy as jnp
from jax import lax
from jax.experimental import pallas as pl
from jax.experimental.pallas import tpu as pltpu

N_DEV = 32


def kernel(x):
    m, n = x.shape

    def body(x_ref, out_ref, halo_ref, send_sems, recv_sems):
        my = lax.axis_index("i")
        left = lax.rem(my + N_DEV - 1, N_DEV)
        right = lax.rem(my + 1, N_DEV)
        has_left = my > 0
        has_right = my < N_DEV - 1

        barrier_sem = pltpu.get_barrier_semaphore()

        @pl.when(has_left)
        def _():
            pl.semaphore_signal(
                barrier_sem, inc=1,
                device_id=(left,), device_id_type=pl.DeviceIdType.MESH,
            )

        @pl.when(has_right)
        def _():
            pl.semaphore_signal(
                barrier_sem, inc=1,
                device_id=(right,), device_id_type=pl.DeviceIdType.MESH,
            )

        @pl.when(has_left)
        def _():
            pl.semaphore_wait(barrier_sem, 1)

        @pl.when(has_right)
        def _():
            pl.semaphore_wait(barrier_sem, 1)

        rdma_r = pltpu.make_async_remote_copy(
            src_ref=x_ref.at[pl.ds(m - 1, 1)],
            dst_ref=halo_ref.at[0],
            send_sem=send_sems.at[0],
            recv_sem=recv_sems.at[0],
            device_id=(right,),
            device_id_type=pl.DeviceIdType.MESH,
        )
        rdma_l = pltpu.make_async_remote_copy(
            src_ref=x_ref.at[pl.ds(0, 1)],
            dst_ref=halo_ref.at[1],
            send_sem=send_sems.at[1],
            recv_sem=recv_sems.at[1],
            device_id=(left,),
            device_id_type=pl.DeviceIdType.MESH,
        )

        @pl.when(has_right)
        def _():
            rdma_r.start()

        @pl.when(has_left)
        def _():
            rdma_l.start()

        xv = x_ref[:, :]
        ri = lax.broadcasted_iota(jnp.int32, (m, m), 0)
        ci = lax.broadcasted_iota(jnp.int32, (m, m), 1)
        band = jnp.where(ri == ci, jnp.float32(0.5), jnp.float32(0.0)) + jnp.where(
            jnp.abs(ri - ci) == 1, jnp.float32(0.25), jnp.float32(0.0)
        )
        out_ref[:, :] = jnp.dot(band, xv, preferred_element_type=jnp.float32)

        @pl.when(has_left)
        def _():
            rdma_r.wait_recv()
            out_ref[pl.ds(0, 1), :] = (
                0.25 * halo_ref[0] + 0.5 * xv[:1] + 0.25 * xv[1:2]
            )

        @pl.when(jnp.logical_not(has_left))
        def _():
            out_ref[pl.ds(0, 1), :] = xv[:1]

        @pl.when(has_right)
        def _():
            rdma_l.wait_recv()
            out_ref[pl.ds(m - 1, 1), :] = (
                0.25 * xv[m - 2:m - 1] + 0.5 * xv[m - 1:] + 0.25 * halo_ref[1]
            )

        @pl.when(jnp.logical_not(has_right))
        def _():
            out_ref[pl.ds(m - 1, 1), :] = xv[m - 1:]

        @pl.when(has_right)
        def _():
            rdma_r.wait_send()

        @pl.when(has_left)
        def _():
            rdma_l.wait_send()

    return pl.pallas_call(
        body,
        out_shape=jax.ShapeDtypeStruct((m, n), x.dtype),
        in_specs=[pl.BlockSpec(memory_space=pltpu.VMEM)],
        out_specs=pl.BlockSpec(memory_space=pltpu.VMEM),
        scratch_shapes=[
            pltpu.VMEM((2, 1, n), x.dtype),
            pltpu.SemaphoreType.DMA((2,)),
            pltpu.SemaphoreType.DMA((2,)),
        ],
        compiler_params=pltpu.CompilerParams(collective_id=0),
    )(x)
